# baseline (speedup 1.0000x reference)
import numpy as np

import jax
import jax.numpy as jnp
from jax import lax
from jax.experimental import pallas as pl
from jax.experimental.pallas import tpu as pltpu
from jax.experimental.pallas import tpu_sc as plsc

_N = 262144
_C = 128
_R = _N // _C
_D = 64
_LOGN = 18

_NC = 2
_NS = 16
_NW = _NC * _NS
_CH = 128


def _lex_gt(ap, ai, bp, bi):
    return (ap > bp) | ((ap == bp) & (ai > bi))


def _sort_body(phit_ref, sorted_ref, idx_ref, dphi_ref, didx_ref):
    r_io = lax.broadcasted_iota(jnp.int32, (_R, _C), 0)
    c_io = lax.broadcasted_iota(jnp.int32, (_R, _C), 1)

    phi = jnp.transpose(phit_ref[...])
    idx = c_io * _R + r_io

    _B = 1024

    def row_stage(phi, idx, s, down, aligned=False):
        d = jnp.int32(1) << s
        if aligned:
            d = pl.multiple_of(d, 8)
        dphi_ref[_B:_B + _R] = phi
        didx_ref[_B:_B + _R] = idx
        up = dphi_ref[pl.ds(_B + d, _R)]
        vp = dphi_ref[pl.ds(_B - d, _R)]
        ui = didx_ref[pl.ds(_B + d, _R)]
        vi = didx_ref[pl.ds(_B - d, _R)]
        is_b = ((r_io >> s) & 1) == 1
        pp = jnp.where(is_b, vp, up)
        pi = jnp.where(is_b, vi, ui)
        gt = _lex_gt(phi, idx, pp, pi)
        take = gt ^ down ^ is_b
        return jnp.where(take, pp, phi), jnp.where(take, pi, idx)

    def phase1(k, carry):
        phi, idx = carry
        down = (jnp.where(k <= 10, (r_io >> k) & 1, c_io & 1)) == 1

        def st_a(t, c2):
            return row_stage(*c2, k - 1 - t, down, aligned=True)

        def st_u(t, c2):
            return row_stage(*c2, jnp.minimum(k, 3) - 1 - t, down)

        carry = lax.fori_loop(0, jnp.maximum(k - 3, 0), st_a, carry)
        return lax.fori_loop(0, jnp.minimum(k, 3), st_u, carry)

    phi, idx = lax.fori_loop(1, 12, phase1, (phi, idx))

    def phase2(k, carry):
        phi, idx = carry
        down = ((c_io >> (k - 11)) & 1) == 1
        for u in range(6, -1, -1):
            dist = 1 << u
            is_b = ((c_io >> u) & 1) == 1
            fwd_p = pltpu.roll(phi, dist, 1)
            bwd_p = pltpu.roll(phi, _C - dist, 1)
            fwd_i = pltpu.roll(idx, dist, 1)
            bwd_i = pltpu.roll(idx, _C - dist, 1)
            pp = jnp.where(is_b, fwd_p, bwd_p)
            pi = jnp.where(is_b, fwd_i, bwd_i)
            gt = _lex_gt(phi, idx, pp, pi)
            take = (gt ^ down ^ is_b) & (u <= k - 12)
            phi = jnp.where(take, pp, phi)
            idx = jnp.where(take, pi, idx)

        def st_a(t, c2):
            return row_stage(*c2, 10 - t, down, aligned=True)

        def st_u(t, c2):
            return row_stage(*c2, 2 - t, down)

        carry = lax.fori_loop(0, 8, st_a, (phi, idx))
        return lax.fori_loop(0, 3, st_u, carry)

    phi, idx = lax.fori_loop(12, _LOGN + 1, phase2, (phi, idx))
    sorted_ref[...] = jnp.transpose(phi)
    idx_ref[...] = jnp.transpose(idx)


def _sort(phi_t):
    return pl.pallas_call(
        _sort_body,
        out_shape=[
            jax.ShapeDtypeStruct((_C, _R), jnp.float32),
            jax.ShapeDtypeStruct((_C, _R), jnp.int32),
        ],
        scratch_shapes=[
            pltpu.VMEM((2 * _R, _C), jnp.float32),
            pltpu.VMEM((2 * _R, _C), jnp.int32),
        ],
    )(phi_t)


def _gather_body(emb_hbm, idx_hbm, out_hbm, idx_v, rows_v, sem):
    wid = lax.axis_index("s") * _NC + lax.axis_index("c")
    n_chunks = _N // (_NW * _CH)
    pltpu.sync_copy(idx_hbm.at[pl.ds(wid * n_chunks, n_chunks)], idx_v)

    def step(q, carry):
        pltpu.async_copy(emb_hbm.at[idx_v.at[q]], rows_v, sem).wait()
        row0 = (wid * n_chunks + q) * _CH
        pltpu.sync_copy(rows_v, out_hbm.at[pl.ds(row0, _CH)])
        return carry

    lax.fori_loop(0, n_chunks, step, 0)


def _gather(emb, idx2):
    n_chunks = _N // (_NW * _CH)
    mesh = plsc.VectorSubcoreMesh(core_axis_name="c", subcore_axis_name="s")
    f = pl.kernel(
        _gather_body,
        out_type=jax.ShapeDtypeStruct((_N, _D), jnp.float32),
        mesh=mesh,
        compiler_params=pltpu.CompilerParams(use_tc_tiling_on_sc=False),
        scratch_types=[
            pltpu.VMEM((n_chunks, _CH), jnp.int32),
            pltpu.VMEM((_CH, _D), jnp.float32),
            pltpu.SemaphoreType.DMA,
        ],
    )
    return f(emb, idx2)


def kernel(key_phi, key_embed):
    phi_t = key_phi.reshape(_C, _R)
    sorted_t, idx_t = _sort(phi_t)
    idx2 = idx_t.reshape(_N).reshape(_R, _C)
    emb = key_embed.reshape(_N, _D)
    out = _gather(emb, idx2)
    return (sorted_t.reshape(1, _N), out.reshape(1, _N, _D))


# speedup vs baseline: 1.0493x; 1.0493x over previous
import numpy as np

import jax
import jax.numpy as jnp
from jax import lax
from jax.experimental import pallas as pl
from jax.experimental.pallas import tpu as pltpu
from jax.experimental.pallas import tpu_sc as plsc

_N = 262144
_C = 128
_R = _N // _C
_D = 64
_LOGN = 18

_NC = 2
_NS = 16
_NW = _NC * _NS
_CH = 128


def _lex_gt(ap, ai, bp, bi):
    return (ap > bp) | ((ap == bp) & (ai > bi))


def _sort_body(phit_ref, sorted_ref, idx_ref, dphi_ref, didx_ref):
    r_io = lax.broadcasted_iota(jnp.int32, (_R, _C), 0)
    c_io = lax.broadcasted_iota(jnp.int32, (_R, _C), 1)

    phi = jnp.transpose(phit_ref[...])
    idx = c_io * _R + r_io

    _B = 1024

    def row_stage(phi, idx, s, down, aligned=False):
        d = jnp.int32(1) << s
        if aligned:
            d = pl.multiple_of(d, 8)
        dphi_ref[_B:_B + _R] = phi
        didx_ref[_B:_B + _R] = idx
        up = dphi_ref[pl.ds(_B + d, _R)]
        vp = dphi_ref[pl.ds(_B - d, _R)]
        ui = didx_ref[pl.ds(_B + d, _R)]
        vi = didx_ref[pl.ds(_B - d, _R)]
        is_b = ((r_io >> s) & 1) == 1
        pp = jnp.where(is_b, vp, up)
        pi = jnp.where(is_b, vi, ui)
        gt = _lex_gt(phi, idx, pp, pi)
        take = gt ^ down ^ is_b
        return jnp.where(take, pp, phi), jnp.where(take, pi, idx)

    def phase1(k, carry):
        phi, idx = carry
        down = (jnp.where(k <= 10, (r_io >> k) & 1, c_io & 1)) == 1

        def st_a(t, c2):
            return row_stage(*c2, k - 1 - t, down, aligned=True)

        def st_u(t, c2):
            return row_stage(*c2, jnp.minimum(k, 3) - 1 - t, down)

        carry = lax.fori_loop(0, jnp.maximum(k - 3, 0), st_a, carry)
        return lax.fori_loop(0, jnp.minimum(k, 3), st_u, carry)

    phi, idx = lax.fori_loop(1, 12, phase1, (phi, idx))

    def phase2(k, carry):
        phi, idx = carry
        down = ((c_io >> (k - 11)) & 1) == 1
        for u in range(6, -1, -1):
            dist = 1 << u
            is_b = ((c_io >> u) & 1) == 1
            fwd_p = pltpu.roll(phi, dist, 1)
            bwd_p = pltpu.roll(phi, _C - dist, 1)
            fwd_i = pltpu.roll(idx, dist, 1)
            bwd_i = pltpu.roll(idx, _C - dist, 1)
            pp = jnp.where(is_b, fwd_p, bwd_p)
            pi = jnp.where(is_b, fwd_i, bwd_i)
            gt = _lex_gt(phi, idx, pp, pi)
            take = (gt ^ down ^ is_b) & (u <= k - 12)
            phi = jnp.where(take, pp, phi)
            idx = jnp.where(take, pi, idx)

        def st_a(t, c2):
            return row_stage(*c2, 10 - t, down, aligned=True)

        def st_u(t, c2):
            return row_stage(*c2, 2 - t, down)

        carry = lax.fori_loop(0, 8, st_a, (phi, idx))
        return lax.fori_loop(0, 3, st_u, carry)

    phi, idx = lax.fori_loop(12, _LOGN + 1, phase2, (phi, idx))
    sorted_ref[...] = jnp.transpose(phi)
    idx_ref[...] = jnp.transpose(idx)


def _sort(phi_t):
    return pl.pallas_call(
        _sort_body,
        out_shape=[
            jax.ShapeDtypeStruct((_C, _R), jnp.float32),
            jax.ShapeDtypeStruct((_C, _R), jnp.int32),
        ],
        scratch_shapes=[
            pltpu.VMEM((2 * _R, _C), jnp.float32),
            pltpu.VMEM((2 * _R, _C), jnp.int32),
        ],
    )(phi_t)


def _gather_body(emb_hbm, idx_hbm, out_hbm, idx_v, rows0, rows1, sem0, sem1):
    wid = lax.axis_index("s") * _NC + lax.axis_index("c")
    n_chunks = _N // (_NW * _CH)
    base = wid * n_chunks
    pltpu.sync_copy(idx_hbm.at[pl.ds(base, n_chunks)], idx_v)

    pltpu.async_copy(emb_hbm.at[idx_v.at[0]], rows0, sem0)

    def step(g, carry):
        q0 = 2 * g
        q1 = q0 + 1
        pltpu.async_copy(emb_hbm.at[idx_v.at[q1]], rows1, sem1)
        pltpu.make_async_copy(emb_hbm.at[idx_v.at[q0]], rows0, sem0).wait()
        pltpu.sync_copy(rows0, out_hbm.at[pl.ds((base + q0) * _CH, _CH)])

        @pl.when(g < n_chunks // 2 - 1)
        def _():
            pltpu.async_copy(emb_hbm.at[idx_v.at[q0 + 2]], rows0, sem0)

        pltpu.make_async_copy(emb_hbm.at[idx_v.at[q1]], rows1, sem1).wait()
        pltpu.sync_copy(rows1, out_hbm.at[pl.ds((base + q1) * _CH, _CH)])
        return carry

    lax.fori_loop(0, n_chunks // 2, step, 0)


def _gather(emb, idx2):
    n_chunks = _N // (_NW * _CH)
    mesh = plsc.VectorSubcoreMesh(core_axis_name="c", subcore_axis_name="s")
    f = pl.kernel(
        _gather_body,
        out_type=jax.ShapeDtypeStruct((_N, _D), jnp.float32),
        mesh=mesh,
        compiler_params=pltpu.CompilerParams(use_tc_tiling_on_sc=False),
        scratch_types=[
            pltpu.VMEM((n_chunks, _CH), jnp.int32),
            pltpu.VMEM((_CH, _D), jnp.float32),
            pltpu.VMEM((_CH, _D), jnp.float32),
            pltpu.SemaphoreType.DMA,
            pltpu.SemaphoreType.DMA,
        ],
    )
    return f(emb, idx2)


def kernel(key_phi, key_embed):
    phi_t = key_phi.reshape(_C, _R)
    sorted_t, idx_t = _sort(phi_t)
    idx2 = idx_t.reshape(_N).reshape(_R, _C)
    emb = key_embed.reshape(_N, _D)
    out = _gather(emb, idx2)
    return (sorted_t.reshape(1, _N), out.reshape(1, _N, _D))
